# baseline (speedup 1.0000x reference)
import functools

import jax
import jax.numpy as jnp
from jax import lax
from jax.experimental import pallas as pl
from jax.experimental.pallas import tpu as pltpu
from jax.experimental.pallas import tpu_sc as plsc

HIDDEN = 768
E = 8
TOP_K = 2
NEG = -1e30

BATCH = 4
SEQ = 8192
N_TOK = BATCH * SEQ
NW = 32
CHUNK = N_TOK // NW
L = 16
STEPS = CHUNK // L


def _logits_kernel(x_ref, w_ref, lg_ref, aux_ref, acc_ref, *, n_tokens):
    i = pl.program_id(0)
    nblk = pl.num_programs(0)

    x = x_ref[...]
    w = w_ref[...]
    logits = jax.lax.dot_general(
        w, x, (((1,), (1,)), ((), ())),
        preferred_element_type=jnp.float32)
    lg_ref[...] = logits

    iota = lax.broadcasted_iota(jnp.int32, logits.shape, 0)
    m1 = jnp.max(logits, axis=0, keepdims=True)
    idx1 = jnp.min(jnp.where(logits == m1, iota, E), axis=0, keepdims=True)

    ex = jnp.exp(logits - m1)
    probs = ex / jnp.sum(ex, axis=0, keepdims=True)
    psum = jnp.sum(probs, axis=1, keepdims=True)
    csum = jnp.sum((iota == idx1).astype(jnp.float32),
                   axis=1, keepdims=True)

    @pl.when(i == 0)
    def _init():
        acc_ref[...] = jnp.zeros_like(acc_ref)

    acc_ref[:, 0:1] += psum
    acc_ref[:, 1:2] += csum

    @pl.when(i == nblk - 1)
    def _final():
        scale = E / float(n_tokens * n_tokens)
        aux_ref[...] = (scale * jnp.sum(acc_ref[:, 0:1] * acc_ref[:, 1:2])
                        ).reshape(1, 1)


def _tc_logits(xf, W, n_tokens, blk=1024):
    body = functools.partial(_logits_kernel, n_tokens=n_tokens)
    return pl.pallas_call(
        body,
        grid=(n_tokens // blk,),
        in_specs=[
            pl.BlockSpec((blk, HIDDEN), lambda i: (i, 0)),
            pl.BlockSpec((E, HIDDEN), lambda i: (0, 0)),
        ],
        out_specs=[
            pl.BlockSpec((E, blk), lambda i: (0, i)),
            pl.BlockSpec((1, 1), lambda i: (0, 0)),
        ],
        out_shape=[
            jax.ShapeDtypeStruct((E, n_tokens), jnp.float32),
            jax.ShapeDtypeStruct((1, 1), jnp.float32),
        ],
        scratch_shapes=[pltpu.VMEM((E, 2), jnp.float32)],
    )(xf, W)


GRP = N_TOK // 128
GPW = GRP // NW


def _sc_router_body(lg_hbm, rw_hbm, se_hbm, lg_v, rw_b, se_b):
    c = lax.axis_index("c")
    s = lax.axis_index("s")
    wid = s * 2 + c
    base = wid * CHUNK
    pltpu.sync_copy(lg_hbm.at[:, pl.ds(base, CHUNK)], lg_v)

    def step(j, carry):
        sl = pl.ds(j * L, L)
        l = [lg_v[e, sl] for e in range(E)]
        m1 = l[0]
        i1 = jnp.zeros((L,), jnp.int32)
        for e in range(1, E):
            b = l[e] > m1
            m1 = jnp.where(b, l[e], m1)
            i1 = jnp.where(b, e, i1)
        m2 = jnp.full((L,), NEG, jnp.float32)
        i2 = jnp.zeros((L,), jnp.int32)
        for e in range(E):
            b = (l[e] > m2) & (i1 != e)
            m2 = jnp.where(b, l[e], m2)
            i2 = jnp.where(b, e, i2)
        e2 = jnp.exp(m2 - m1)
        d = 1.0 + e2
        g = j // (128 // L)
        r = pl.ds((j % (128 // L)) * L, L)
        rw_b[g, 0, r] = 1.0 / d
        rw_b[g, 1, r] = e2 / d
        se_b[g, 0, r] = i1
        se_b[g, 1, r] = i2
        return carry

    lax.fori_loop(0, STEPS, step, 0)
    pltpu.sync_copy(rw_b, rw_hbm.at[pl.ds(wid * GPW, GPW)])
    pltpu.sync_copy(se_b, se_hbm.at[pl.ds(wid * GPW, GPW)])


@functools.cache
def _sc_router():
    return pl.kernel(
        _sc_router_body,
        out_type=[
            jax.ShapeDtypeStruct((GRP, TOP_K, 128), jnp.float32),
            jax.ShapeDtypeStruct((GRP, TOP_K, 128), jnp.int32),
        ],
        mesh=plsc.VectorSubcoreMesh(core_axis_name="c", subcore_axis_name="s",
                                    num_cores=2, num_subcores=16),
        scratch_types=[
            pltpu.VMEM((E, CHUNK), jnp.float32),
            pltpu.VMEM((GPW, TOP_K, 128), jnp.float32),
            pltpu.VMEM((GPW, TOP_K, 128), jnp.int32),
        ],
        compiler_params=pltpu.CompilerParams(needs_layout_passes=False),
    )


def kernel(x, W):
    B, S, H = x.shape
    n_tokens = B * S
    xf = x.reshape(n_tokens, H)
    lg, aux = _tc_logits(xf, W, n_tokens)
    rw3, se3 = _sc_router()(lg)
    rw = rw3.transpose(0, 2, 1).reshape(B, S, TOP_K)
    se = se3.transpose(0, 2, 1).reshape(B, S, TOP_K)
    return (rw, se, aux[0, 0])


# speedup vs baseline: 1.8670x; 1.8670x over previous
import functools

import jax
import jax.numpy as jnp
from jax import lax
from jax.experimental import pallas as pl
from jax.experimental.pallas import tpu as pltpu

HIDDEN = 768
E = 8
TOP_K = 2
NEG = -1e30


def _router_kernel(x_ref, w_ref, rw_ref, se_ref, aux_ref, acc_ref, *, n_tokens):
    i = pl.program_id(0)
    nblk = pl.num_programs(0)

    x = x_ref[...]
    w = w_ref[...]
    logits = lax.dot_general(
        w, x, (((1,), (1,)), ((), ())),
        preferred_element_type=jnp.float32)

    iota = lax.broadcasted_iota(jnp.int32, logits.shape, 0)
    m1 = jnp.max(logits, axis=0, keepdims=True)
    i1 = jnp.min(jnp.where(logits == m1, iota, E), axis=0, keepdims=True)
    masked = jnp.where(iota == i1, NEG, logits)
    m2 = jnp.max(masked, axis=0, keepdims=True)
    i2 = jnp.min(jnp.where(masked == m2, iota, E), axis=0, keepdims=True)

    w1 = 1.0 / (1.0 + jnp.exp(m2 - m1))
    rw_ref[...] = jnp.concatenate([w1, 1.0 - w1], axis=0)
    se_ref[...] = jnp.concatenate([i1, i2], axis=0)

    ex = jnp.exp(logits - m1)
    probs = ex * (1.0 / jnp.sum(ex, axis=0, keepdims=True))
    psum = jnp.sum(probs, axis=1, keepdims=True)
    csum = jnp.sum((iota == i1).astype(jnp.float32),
                   axis=1, keepdims=True)

    @pl.when(i == 0)
    def _init():
        acc_ref[...] = jnp.zeros_like(acc_ref)

    acc_ref[:, 0:1] += psum
    acc_ref[:, 1:2] += csum

    @pl.when(i == nblk - 1)
    def _final():
        scale = E / float(n_tokens * n_tokens)
        aux_ref[...] = (scale * jnp.sum(acc_ref[:, 0:1] * acc_ref[:, 1:2])
                        ).reshape(1, 1)


def kernel(x, W):
    B, S, H = x.shape
    n_tokens = B * S
    xf = x.reshape(n_tokens, H)
    BLK = 1024
    grid = (n_tokens // BLK,)

    body = functools.partial(_router_kernel, n_tokens=n_tokens)

    rw_t, se_t, aux = pl.pallas_call(
        body,
        grid=grid,
        in_specs=[
            pl.BlockSpec((BLK, H), lambda i: (i, 0)),
            pl.BlockSpec((E, H), lambda i: (0, 0)),
        ],
        out_specs=[
            pl.BlockSpec((TOP_K, BLK), lambda i: (0, i)),
            pl.BlockSpec((TOP_K, BLK), lambda i: (0, i)),
            pl.BlockSpec((1, 1), lambda i: (0, 0)),
        ],
        out_shape=[
            jax.ShapeDtypeStruct((TOP_K, n_tokens), jnp.float32),
            jax.ShapeDtypeStruct((TOP_K, n_tokens), jnp.int32),
            jax.ShapeDtypeStruct((1, 1), jnp.float32),
        ],
        scratch_shapes=[pltpu.VMEM((E, 2), jnp.float32)],
    )(xf, W)

    rw = rw_t.T.reshape(B, S, TOP_K)
    se = se_t.T.reshape(B, S, TOP_K)
    return (rw, se, aux[0, 0])
